# baseline (speedup 1.0000x reference)
import functools

import jax
import jax.numpy as jnp
from jax import lax
from jax.experimental import pallas as pl
from jax.experimental.pallas import tpu as pltpu
from jax.experimental.pallas import tpu_sc as plsc

N = 10000
D = 128
E = 320000

NC = 2
NS = 16
NW = NC * NS

K = 128
NB = 16
NBLK = 5
CH = NB * NBLK
EPT = CH * K
E_PAD = EPT * NW
ROWS = E_PAD // K

NP = 10240
RPS = NP // NS

_sc_mesh = plsc.VectorSubcoreMesh(core_axis_name="c", subcore_axis_name="s")


@functools.partial(
    pl.kernel,
    out_type=(
        jax.ShapeDtypeStruct((NC * NP, D), jnp.float32),
        jax.ShapeDtypeStruct((NW, NP), jnp.float32),
    ),
    mesh=_sc_mesh,
    scratch_types=[
        pltpu.VMEM((K,), jnp.int32),
        pltpu.VMEM((K,), jnp.int32),
        pltpu.VMEM((K,), jnp.int32),
        pltpu.VMEM((K,), jnp.int32),
        pltpu.VMEM((K, D), jnp.float32),
        pltpu.VMEM((K, D), jnp.float32),
        pltpu.VMEM((NP,), jnp.float32),
        pltpu.VMEM_SHARED((NP, D), jnp.float32),
        pltpu.SemaphoreType.DMA,
        pltpu.SemaphoreType.DMA,
    ],
    compiler_params=pltpu.CompilerParams(needs_layout_passes=False),
)
def _sc_segsum(h_hbm, src_hbm, dst_hbm, z_d_hbm, z_deg_hbm,
               out_sum, out_deg,
               src_a, dst_a, src_b, dst_b, rows_a, rows_b, deg_v, acc,
               sem_a, sem_b):
    cid = lax.axis_index("c")
    sid = lax.axis_index("s")
    wid = sid * NC + cid

    pltpu.sync_copy(z_d_hbm, rows_a)
    pltpu.sync_copy(z_deg_hbm, deg_v)

    @pl.loop(0, RPS // K)
    def zinit(j):
        pltpu.sync_copy(rows_a, acc.at[pl.ds(sid * RPS + j * K, K)])

    plsc.subcore_barrier()

    ones16 = jnp.ones((16,), jnp.float32)

    def deg_update(dst_v):
        for l in range(K // 16):
            plsc.addupdate_scatter(deg_v, [dst_v[pl.ds(l * 16, 16)]], ones16)

    @pl.loop(0, CH)
    def chunk(i):
        off = wid * EPT + i * K
        pltpu.sync_copy(src_hbm.at[pl.ds(off, K)], src_a)
        pltpu.sync_copy(dst_hbm.at[pl.ds(off, K)], dst_a)
        pltpu.async_copy(h_hbm.at[src_a], rows_a, sem_a).wait()
        pltpu.sync_copy(rows_a, acc.at[dst_a], add=True)

    plsc.subcore_barrier()

    @pl.loop(0, RPS // K)
    def zout(j):
        r = sid * RPS + j * K
        pltpu.sync_copy(acc.at[pl.ds(r, K)], rows_a)
        pltpu.sync_copy(rows_a, out_sum.at[pl.ds(cid * NP + r, K)])

    pltpu.sync_copy(deg_v, out_deg.at[wid])


def _relu_body(x_ref, o_ref):
    o_ref[...] = jnp.maximum(x_ref[...], 0.0)


def _relu_tc(x):
    rows = 1000
    return pl.pallas_call(
        _relu_body,
        out_shape=jax.ShapeDtypeStruct((N, D), jnp.float32),
        grid=(N // rows,),
        in_specs=[pl.BlockSpec((rows, D), lambda i: (i, 0))],
        out_specs=pl.BlockSpec((rows, D), lambda i: (i, 0)),
    )(x)


def _combine_body(h_ref, s_ref, d_ref, ws_ref, wn_ref, b_ref, o_ref, *, relu):
    summed = s_ref[0] + s_ref[1]
    deg = jnp.sum(d_ref[...], axis=1)[:, None]
    agg = summed / jnp.maximum(deg, 1.0)
    out = (jnp.dot(h_ref[...], ws_ref[...], preferred_element_type=jnp.float32)
           + jnp.dot(agg, wn_ref[...], preferred_element_type=jnp.float32)
           + b_ref[...])
    if relu:
        out = jnp.maximum(out, 0.0)
    o_ref[...] = out


def _combine_tc(h, s, d, w_self, w_neigh, b, relu):
    rows = 1000
    return pl.pallas_call(
        functools.partial(_combine_body, relu=relu),
        out_shape=jax.ShapeDtypeStruct((N, D), jnp.float32),
        grid=(N // rows,),
        in_specs=[
            pl.BlockSpec((rows, D), lambda i: (i, 0)),
            pl.BlockSpec((NC, rows, D), lambda i: (0, i, 0)),
            pl.BlockSpec((rows, NW), lambda i: (i, 0)),
            pl.BlockSpec((D, D), lambda i: (0, 0)),
            pl.BlockSpec((D, D), lambda i: (0, 0)),
            pl.BlockSpec((D,), lambda i: (0,)),
        ],
        out_specs=pl.BlockSpec((rows, D), lambda i: (i, 0)),
    )(h, s, d, w_self, w_neigh, b)


def _pad_edges(edge_index):
    pad = E_PAD - E
    src = jnp.concatenate([edge_index[0], jnp.zeros((pad,), jnp.int32)])
    dst = jnp.concatenate([edge_index[1], jnp.full((pad,), N, jnp.int32)])
    return src, dst


def kernel(feat, edge_index1, edge_index2, W_self1, W_neigh1, b1,
           W_self2, W_neigh2, b2):
    src1, dst1 = _pad_edges(edge_index1)
    src2, dst2 = _pad_edges(edge_index2)
    z_d = jnp.zeros((K, D), jnp.float32)
    z_deg = jnp.zeros((NP,), jnp.float32)

    h0 = _relu_tc(feat)
    s1, d1 = _sc_segsum(h0, src1, dst1, z_d, z_deg)
    h1 = _combine_tc(h0, s1.reshape(NC, NP, D), d1.T,
                     W_self1, W_neigh1, b1, relu=True)
    s2, d2 = _sc_segsum(h1, src2, dst2, z_d, z_deg)
    out = _combine_tc(h1, s2.reshape(NC, NP, D), d2.T,
                      W_self2, W_neigh2, b2, relu=False)
    return out


# speedup vs baseline: 1.3948x; 1.3948x over previous
import functools

import jax
import jax.numpy as jnp
from jax import lax
from jax.experimental import pallas as pl
from jax.experimental.pallas import tpu as pltpu
from jax.experimental.pallas import tpu_sc as plsc

N = 10000
D = 128
E = 320000

NC = 2
NS = 16
NW = NC * NS

K = 128
CH = -(-E // (NW * K))
EPT = CH * K
E_PAD = EPT * NW

NP = 10240
RPS = NP // NS

_sc_mesh = plsc.VectorSubcoreMesh(core_axis_name="c", subcore_axis_name="s")


@functools.partial(
    pl.kernel,
    out_type=(
        jax.ShapeDtypeStruct((NC * NP, D), jnp.float32),
        jax.ShapeDtypeStruct((NW, NP), jnp.float32),
    ),
    mesh=_sc_mesh,
    scratch_types=[
        pltpu.VMEM((K,), jnp.int32),
        pltpu.VMEM((K,), jnp.int32),
        pltpu.VMEM((K, D), jnp.float32),
        pltpu.VMEM((NP,), jnp.float32),
        pltpu.VMEM_SHARED((NP, D), jnp.float32),
        pltpu.SemaphoreType.DMA,
    ],
    compiler_params=pltpu.CompilerParams(needs_layout_passes=False),
)
def _sc_segsum(h_hbm, src_hbm, dst_hbm, z_d_hbm, z_deg_hbm,
               out_sum, out_deg,
               src_v, dst_v, rows_v, deg_v, acc, sem):
    cid = lax.axis_index("c")
    sid = lax.axis_index("s")
    wid = sid * NC + cid

    pltpu.sync_copy(z_d_hbm, rows_v)
    pltpu.sync_copy(z_deg_hbm, deg_v)

    @pl.loop(0, RPS // K)
    def zinit(j):
        pltpu.sync_copy(rows_v, acc.at[pl.ds(sid * RPS + j * K, K)])

    plsc.subcore_barrier()

    ones16 = jnp.ones((16,), jnp.float32)

    def deg_update(dst_v):
        for l in range(K // 16):
            plsc.addupdate_scatter(deg_v, [dst_v[pl.ds(l * 16, 16)]], ones16)

    @pl.loop(0, CH)
    def chunk(i):
        off = wid * EPT + i * K
        pltpu.sync_copy(src_hbm.at[pl.ds(off, K)], src_v)
        pltpu.sync_copy(dst_hbm.at[pl.ds(off, K)], dst_v)
        pltpu.async_copy(h_hbm.at[src_v], rows_v, sem).wait()
        pltpu.sync_copy(rows_v, acc.at[dst_v], add=True)
        deg_update(dst_v)

    plsc.subcore_barrier()

    @pl.loop(0, RPS // K)
    def zout(j):
        r = sid * RPS + j * K
        pltpu.sync_copy(acc.at[pl.ds(r, K)], rows_v)
        pltpu.sync_copy(rows_v, out_sum.at[pl.ds(cid * NP + r, K)])

    pltpu.sync_copy(deg_v, out_deg.at[wid])


def _relu_body(x_ref, o_ref):
    o_ref[...] = jnp.maximum(x_ref[...], 0.0)


def _relu_tc(x):
    rows = 1000
    return pl.pallas_call(
        _relu_body,
        out_shape=jax.ShapeDtypeStruct((N, D), jnp.float32),
        grid=(N // rows,),
        in_specs=[pl.BlockSpec((rows, D), lambda i: (i, 0))],
        out_specs=pl.BlockSpec((rows, D), lambda i: (i, 0)),
    )(x)


def _combine_body(h_ref, s_ref, d_ref, ws_ref, wn_ref, b_ref, o_ref, *, relu):
    summed = s_ref[0] + s_ref[1]
    deg = jnp.sum(d_ref[...], axis=1)[:, None]
    agg = summed / jnp.maximum(deg, 1.0)
    out = (jnp.dot(h_ref[...], ws_ref[...], preferred_element_type=jnp.float32)
           + jnp.dot(agg, wn_ref[...], preferred_element_type=jnp.float32)
           + b_ref[...])
    if relu:
        out = jnp.maximum(out, 0.0)
    o_ref[...] = out


def _combine_tc(h, s, d, w_self, w_neigh, b, relu):
    rows = 1000
    return pl.pallas_call(
        functools.partial(_combine_body, relu=relu),
        out_shape=jax.ShapeDtypeStruct((N, D), jnp.float32),
        grid=(N // rows,),
        in_specs=[
            pl.BlockSpec((rows, D), lambda i: (i, 0)),
            pl.BlockSpec((NC, rows, D), lambda i: (0, i, 0)),
            pl.BlockSpec((rows, NW), lambda i: (i, 0)),
            pl.BlockSpec((D, D), lambda i: (0, 0)),
            pl.BlockSpec((D, D), lambda i: (0, 0)),
            pl.BlockSpec((D,), lambda i: (0,)),
        ],
        out_specs=pl.BlockSpec((rows, D), lambda i: (i, 0)),
    )(h, s, d, w_self, w_neigh, b)


def _pad_edges(edge_index):
    pad = E_PAD - E
    src = jnp.concatenate([edge_index[0], jnp.zeros((pad,), jnp.int32)])
    dst = jnp.concatenate([edge_index[1], jnp.full((pad,), N, jnp.int32)])
    return src, dst


def kernel(feat, edge_index1, edge_index2, W_self1, W_neigh1, b1,
           W_self2, W_neigh2, b2):
    src1, dst1 = _pad_edges(edge_index1)
    src2, dst2 = _pad_edges(edge_index2)
    z_d = jnp.zeros((K, D), jnp.float32)
    z_deg = jnp.zeros((NP,), jnp.float32)

    h0 = _relu_tc(feat)
    s1, d1 = _sc_segsum(h0, src1, dst1, z_d, z_deg)
    h1 = _combine_tc(h0, s1.reshape(NC, NP, D), d1.T,
                     W_self1, W_neigh1, b1, relu=True)
    s2, d2 = _sc_segsum(h1, src2, dst2, z_d, z_deg)
    out = _combine_tc(h1, s2.reshape(NC, NP, D), d2.T,
                      W_self2, W_neigh2, b2, relu=False)
    return out


# speedup vs baseline: 1.4909x; 1.0688x over previous
import functools

import jax
import jax.numpy as jnp
from jax import lax
from jax.experimental import pallas as pl
from jax.experimental.pallas import tpu as pltpu
from jax.experimental.pallas import tpu_sc as plsc

N = 10000
D = 128
E = 320000

NC = 2
NS = 16
NW = NC * NS

K = 128
CH = -(-E // (NW * K))
EPT = CH * K
E_PAD = EPT * NW

NP = 10240
RPS = NP // NS

_sc_mesh = plsc.VectorSubcoreMesh(core_axis_name="c", subcore_axis_name="s")


@functools.partial(
    pl.kernel,
    out_type=(
        jax.ShapeDtypeStruct((NC * NP, D), jnp.float32),
        jax.ShapeDtypeStruct((NW, NP), jnp.float32),
    ),
    mesh=_sc_mesh,
    scratch_types=[
        pltpu.VMEM((EPT,), jnp.int32),
        pltpu.VMEM((K,), jnp.int32),
        pltpu.VMEM((K, D), jnp.float32),
        pltpu.VMEM((NP,), jnp.float32),
        pltpu.VMEM_SHARED((NP, D), jnp.float32),
        pltpu.SemaphoreType.DMA,
    ],
    compiler_params=pltpu.CompilerParams(needs_layout_passes=False),
)
def _sc_segsum(h_hbm, src_hbm, dst_hbm, z_d_hbm, z_deg_hbm,
               out_sum, out_deg,
               src_t, dst_v, rows_v, deg_v, acc, sem):
    cid = lax.axis_index("c")
    sid = lax.axis_index("s")
    wid = sid * NC + cid

    pltpu.sync_copy(z_d_hbm, rows_v)
    pltpu.sync_copy(z_deg_hbm, deg_v)

    @pl.loop(0, RPS // K)
    def zinit(j):
        pltpu.sync_copy(rows_v, acc.at[pl.ds(sid * RPS + j * K, K)])

    plsc.subcore_barrier()

    ones16 = jnp.ones((16,), jnp.float32)

    pltpu.sync_copy(src_hbm.at[pl.ds(wid * EPT, EPT)], src_t)

    @pl.loop(0, CH)
    def chunk(i):
        pltpu.sync_copy(dst_hbm.at[pl.ds(wid * EPT + i * K, K)], dst_v)
        pltpu.async_copy(h_hbm.at[src_t.at[pl.ds(i * K, K)]], rows_v, sem).wait()
        pltpu.sync_copy(rows_v, acc.at[dst_v], add=True)
        for l in range(K // 16):
            plsc.addupdate_scatter(deg_v, [dst_v[pl.ds(l * 16, 16)]], ones16)

    plsc.subcore_barrier()

    @pl.loop(0, RPS // K)
    def zout(j):
        r = sid * RPS + j * K
        pltpu.sync_copy(acc.at[pl.ds(r, K)], rows_v)
        pltpu.sync_copy(rows_v, out_sum.at[pl.ds(cid * NP + r, K)])

    pltpu.sync_copy(deg_v, out_deg.at[wid])


def _relu_body(x_ref, o_ref):
    o_ref[...] = jnp.maximum(x_ref[...], 0.0)


def _relu_tc(x):
    rows = 1000
    return pl.pallas_call(
        _relu_body,
        out_shape=jax.ShapeDtypeStruct((N, D), jnp.float32),
        grid=(N // rows,),
        in_specs=[pl.BlockSpec((rows, D), lambda i: (i, 0))],
        out_specs=pl.BlockSpec((rows, D), lambda i: (i, 0)),
    )(x)


def _combine_body(h_ref, s_ref, d_ref, ws_ref, wn_ref, b_ref, o_ref, *, relu):
    summed = s_ref[0] + s_ref[1]
    deg = jnp.sum(d_ref[...], axis=1)[:, None]
    agg = summed / jnp.maximum(deg, 1.0)
    out = (jnp.dot(h_ref[...], ws_ref[...], preferred_element_type=jnp.float32)
           + jnp.dot(agg, wn_ref[...], preferred_element_type=jnp.float32)
           + b_ref[...])
    if relu:
        out = jnp.maximum(out, 0.0)
    o_ref[...] = out


def _combine_tc(h, s, d, w_self, w_neigh, b, relu):
    rows = 1000
    return pl.pallas_call(
        functools.partial(_combine_body, relu=relu),
        out_shape=jax.ShapeDtypeStruct((N, D), jnp.float32),
        grid=(N // rows,),
        in_specs=[
            pl.BlockSpec((rows, D), lambda i: (i, 0)),
            pl.BlockSpec((NC, rows, D), lambda i: (0, i, 0)),
            pl.BlockSpec((rows, NW), lambda i: (i, 0)),
            pl.BlockSpec((D, D), lambda i: (0, 0)),
            pl.BlockSpec((D, D), lambda i: (0, 0)),
            pl.BlockSpec((D,), lambda i: (0,)),
        ],
        out_specs=pl.BlockSpec((rows, D), lambda i: (i, 0)),
    )(h, s, d, w_self, w_neigh, b)


def _pad_edges(edge_index):
    pad = E_PAD - E
    src = jnp.concatenate([edge_index[0], jnp.zeros((pad,), jnp.int32)])
    dst = jnp.concatenate([edge_index[1], jnp.full((pad,), N, jnp.int32)])
    return src, dst


def kernel(feat, edge_index1, edge_index2, W_self1, W_neigh1, b1,
           W_self2, W_neigh2, b2):
    src1, dst1 = _pad_edges(edge_index1)
    src2, dst2 = _pad_edges(edge_index2)
    z_d = jnp.zeros((K, D), jnp.float32)
    z_deg = jnp.zeros((NP,), jnp.float32)

    h0 = _relu_tc(feat)
    s1, d1 = _sc_segsum(h0, src1, dst1, z_d, z_deg)
    h1 = _combine_tc(h0, s1.reshape(NC, NP, D), d1.T,
                     W_self1, W_neigh1, b1, relu=True)
    s2, d2 = _sc_segsum(h1, src2, dst2, z_d, z_deg)
    out = _combine_tc(h1, s2.reshape(NC, NP, D), d2.T,
                      W_self2, W_neigh2, b2, relu=False)
    return out


# speedup vs baseline: 1.6224x; 1.0883x over previous
import functools

import jax
import jax.numpy as jnp
from jax import lax
from jax.experimental import pallas as pl
from jax.experimental.pallas import tpu as pltpu
from jax.experimental.pallas import tpu_sc as plsc

N = 10000
D = 128
E = 320000

NC = 2
NS = 16
NW = NC * NS

K = 128
CH = -(-E // (NW * K))
EPT = CH * K
E_PAD = EPT * NW

NP = 10240
RPS = NP // NS

_sc_mesh = plsc.VectorSubcoreMesh(core_axis_name="c", subcore_axis_name="s")


@functools.partial(
    pl.kernel,
    out_type=(
        jax.ShapeDtypeStruct((NC * NP, D), jnp.float32),
        jax.ShapeDtypeStruct((NW, NP), jnp.float32),
    ),
    mesh=_sc_mesh,
    scratch_types=[
        pltpu.VMEM((EPT,), jnp.int32),
        pltpu.VMEM((K,), jnp.int32),
        pltpu.VMEM((K, D), jnp.float32),
        pltpu.VMEM((NP,), jnp.float32),
        pltpu.VMEM_SHARED((NP, D), jnp.float32),
        pltpu.SemaphoreType.DMA,
    ],
    compiler_params=pltpu.CompilerParams(needs_layout_passes=False),
)
def _sc_segsum(h_hbm, src_hbm, dst_hbm, z_d_hbm, z_deg_hbm,
               out_sum, out_deg,
               src_t, dst_v, rows_v, deg_v, acc, sem):
    cid = lax.axis_index("c")
    sid = lax.axis_index("s")
    wid = sid * NC + cid

    pltpu.sync_copy(z_d_hbm, rows_v)
    pltpu.sync_copy(z_deg_hbm, deg_v)

    @pl.loop(0, RPS // K)
    def zinit(j):
        pltpu.sync_copy(rows_v, acc.at[pl.ds(sid * RPS + j * K, K)])

    plsc.subcore_barrier()

    ones16 = jnp.ones((16,), jnp.float32)

    pltpu.sync_copy(src_hbm.at[pl.ds(wid * EPT, EPT)], src_t)

    @pl.loop(0, CH)
    def chunk(i):
        g = pltpu.async_copy(h_hbm.at[src_t.at[pl.ds(i * K, K)]], rows_v, sem)
        pltpu.sync_copy(dst_hbm.at[pl.ds(wid * EPT + i * K, K)], dst_v)
        g.wait()
        pltpu.sync_copy(rows_v, acc.at[dst_v], add=True)
        for l in range(K // 16):
            plsc.addupdate_scatter(deg_v, [dst_v[pl.ds(l * 16, 16)]], ones16)

    plsc.subcore_barrier()

    @pl.loop(0, RPS // K)
    def zout(j):
        r = sid * RPS + j * K
        pltpu.sync_copy(acc.at[pl.ds(r, K)], rows_v)
        pltpu.sync_copy(rows_v, out_sum.at[pl.ds(cid * NP + r, K)])

    pltpu.sync_copy(deg_v, out_deg.at[wid])


def _relu_body(x_ref, o_ref):
    o_ref[...] = jnp.maximum(x_ref[...], 0.0)


def _relu_tc(x):
    rows = 1000
    return pl.pallas_call(
        _relu_body,
        out_shape=jax.ShapeDtypeStruct((N, D), jnp.float32),
        grid=(N // rows,),
        in_specs=[pl.BlockSpec((rows, D), lambda i: (i, 0))],
        out_specs=pl.BlockSpec((rows, D), lambda i: (i, 0)),
    )(x)


def _combine_body(h_ref, s_ref, d_ref, ws_ref, wn_ref, b_ref, o_ref, *, relu):
    summed = s_ref[0] + s_ref[1]
    deg = jnp.sum(d_ref[...], axis=1)[:, None]
    agg = summed / jnp.maximum(deg, 1.0)
    out = (jnp.dot(h_ref[...], ws_ref[...], preferred_element_type=jnp.float32)
           + jnp.dot(agg, wn_ref[...], preferred_element_type=jnp.float32)
           + b_ref[...])
    if relu:
        out = jnp.maximum(out, 0.0)
    o_ref[...] = out


def _combine_tc(h, s, d, w_self, w_neigh, b, relu):
    rows = 1000
    return pl.pallas_call(
        functools.partial(_combine_body, relu=relu),
        out_shape=jax.ShapeDtypeStruct((N, D), jnp.float32),
        grid=(N // rows,),
        in_specs=[
            pl.BlockSpec((rows, D), lambda i: (i, 0)),
            pl.BlockSpec((NC, rows, D), lambda i: (0, i, 0)),
            pl.BlockSpec((rows, NW), lambda i: (i, 0)),
            pl.BlockSpec((D, D), lambda i: (0, 0)),
            pl.BlockSpec((D, D), lambda i: (0, 0)),
            pl.BlockSpec((D,), lambda i: (0,)),
        ],
        out_specs=pl.BlockSpec((rows, D), lambda i: (i, 0)),
    )(h, s, d, w_self, w_neigh, b)


def _pad_edges(edge_index):
    pad = E_PAD - E
    src = jnp.concatenate([edge_index[0], jnp.zeros((pad,), jnp.int32)])
    dst = jnp.concatenate([edge_index[1], jnp.full((pad,), N, jnp.int32)])
    return src, dst


def kernel(feat, edge_index1, edge_index2, W_self1, W_neigh1, b1,
           W_self2, W_neigh2, b2):
    src1, dst1 = _pad_edges(edge_index1)
    src2, dst2 = _pad_edges(edge_index2)
    z_d = jnp.zeros((K, D), jnp.float32)
    z_deg = jnp.zeros((NP,), jnp.float32)

    h0 = _relu_tc(feat)
    s1, d1 = _sc_segsum(h0, src1, dst1, z_d, z_deg)
    h1 = _combine_tc(h0, s1.reshape(NC, NP, D), d1.T,
                     W_self1, W_neigh1, b1, relu=True)
    s2, d2 = _sc_segsum(h1, src2, dst2, z_d, z_deg)
    out = _combine_tc(h1, s2.reshape(NC, NP, D), d2.T,
                      W_self2, W_neigh2, b2, relu=False)
    return out
